# baseline (speedup 1.0000x reference)
import functools

import jax
import jax.numpy as jnp
from jax import lax
from jax.experimental import pallas as pl
from jax.experimental.pallas import tpu as pltpu
from jax.experimental.pallas import tpu_sc as plsc

HIDDEN = 768
B, S = 4, 2048
LN_EPS = 1e-12
L = 16
NCH = HIDDEN // L
NC, NS = 2, 16
NW = NC * NS
POS_PER_W = S // NW
CHUNK = 32
NSTEP = (POS_PER_W // CHUNK) * B


def _allsum_v(v):
    for k in (1, 2, 4, 8):
        idx = lax.iota(jnp.int32, L) ^ k
        perm = lax.gather(
            v, idx[:, None],
            lax.GatherDimensionNumbers(offset_dims=(),
                                       collapsed_slice_dims=(0,),
                                       start_index_map=(0,)),
            slice_sizes=(1,), unique_indices=True,
            mode=lax.GatherScatterMode.PROMISE_IN_BOUNDS)
        v = v + perm
    return v


def _rsqrt_v(x):
    i = lax.bitcast_convert_type(x, jnp.int32)
    i = jnp.int32(0x5F3759DF) - (i >> 1)
    y = lax.bitcast_convert_type(i, jnp.float32)
    half = x * 0.5
    for _ in range(4):
        y = y * (1.5 - half * y * y)
    return y


_mesh = plsc.VectorSubcoreMesh(core_axis_name="c", subcore_axis_name="s")


@functools.partial(
    pl.kernel,
    mesh=_mesh,
    out_type=jax.ShapeDtypeStruct((B * S, HIDDEN), jnp.float32),
    scratch_types=[
        pltpu.VMEM((POS_PER_W, HIDDEN), jnp.float32),
        pltpu.VMEM((HIDDEN,), jnp.float32),
        pltpu.VMEM((HIDDEN,), jnp.float32),
        pltpu.VMEM((HIDDEN,), jnp.float32),
        pltpu.VMEM((CHUNK,), jnp.int32),
        pltpu.VMEM((CHUNK, HIDDEN), jnp.float32),
        pltpu.SemaphoreType.DMA,
    ],
)
def _emb_sc(ids_hbm, wtab_hbm, pe_hbm, te_hbm, g_hbm, bt_hbm, out_hbm,
            pt_v, te_v, g_v, bt_v, idx_v, x_v, sem):
    wid = lax.axis_index("s") * NC + lax.axis_index("c")
    p0 = wid * POS_PER_W

    pltpu.sync_copy(te_hbm.at[0], te_v)
    pltpu.sync_copy(g_hbm, g_v)
    pltpu.sync_copy(bt_hbm, bt_v)
    pltpu.sync_copy(pe_hbm.at[pl.ds(p0, POS_PER_W)], pt_v)

    def pt_body(p, carry):
        for j in range(NCH):
            sl = pl.ds(j * L, L)
            pt_v[p, sl] = pt_v[p, sl] + te_v[sl]
        return carry

    lax.fori_loop(0, POS_PER_W, pt_body, 0)

    def step(c, carry):
        sc = c // B
        b = c % B
        base = b * S + p0 + sc * CHUNK
        pltpu.sync_copy(ids_hbm.at[pl.ds(base, CHUNK)], idx_v)
        pltpu.async_copy(wtab_hbm.at[idx_v], x_v, sem).wait()
        prow0 = sc * CHUNK

        def row(r, rcarry):
            xs = []
            sv = jnp.zeros((L,), jnp.float32)
            qv = jnp.zeros((L,), jnp.float32)
            for j in range(NCH):
                sl = pl.ds(j * L, L)
                xj = x_v[r, sl] + pt_v[prow0 + r, sl]
                xs.append(xj)
                sv = sv + xj
                qv = qv + xj * xj
            mean_v = _allsum_v(sv) * (1.0 / HIDDEN)
            var_v = _allsum_v(qv) * (1.0 / HIDDEN) - mean_v * mean_v
            inv_v = _rsqrt_v(var_v + LN_EPS)
            for j in range(NCH):
                sl = pl.ds(j * L, L)
                x_v[r, sl] = (xs[j] - mean_v) * (inv_v * g_v[sl]) + bt_v[sl]
            return rcarry

        lax.fori_loop(0, CHUNK, row, 0)
        pltpu.sync_copy(x_v, out_hbm.at[pl.ds(base, CHUNK)])
        return carry

    lax.fori_loop(0, NSTEP, step, 0)


def kernel(input_ids, extended_attention_mask, word_emb, pos_emb, type_emb,
           ln_gamma, ln_beta):
    ids = input_ids.reshape(-1).astype(jnp.int32)
    out = _emb_sc(ids, word_emb, pos_emb, type_emb, ln_gamma, ln_beta)
    return out.reshape(B, S, HIDDEN), extended_attention_mask
